# baseline (speedup 1.0000x reference)
import functools

import jax
import jax.numpy as jnp
from jax import lax
from jax.experimental import pallas as pl
from jax.experimental.pallas import tpu as pltpu
from jax.experimental.pallas import tpu_sc as plsc

_CYCLES = (7, 30, 91, 365)
_C_DIM = 64
_BATCH = 16384
_NC = 2
_NS = 16
_L = 16
_BPW = _BATCH // (_NC * _NS)
_CH = 128
_NCHUNK = _BPW // _CH
_ROWS = tuple(c + 1 for c in _CYCLES)
_OFF = tuple(128 * sum(_ROWS[:i]) for i in range(4))
_TAB_LEN = 128 * sum(_ROWS)
_SLOT = _CH * _C_DIM


def _cae_body(x_hbm, tabs_hbm,
              o0, o1, o2, o3, o4, o5, o6, o7,
              x_v, i0, i1, i2, i3, ltab, sb_a, sb_b, cb_a, cb_b,
              sh_tabs, ws0, ws1):
    outs = (o0, o1, o2, o3, o4, o5, o6, o7)
    idx_refs = (i0, i1, i2, i3)
    sbufs = (sb_a, sb_b)
    cbufs = (cb_a, cb_b)
    wsems = (ws0, ws1)

    c_id = lax.axis_index("c")
    s_id = lax.axis_index("s")
    base = (c_id * _NS + s_id) * _BPW

    @pl.when(s_id == 0)
    def _stage_tables():
        pltpu.sync_copy(tabs_hbm, sh_tabs)

    pltpu.sync_copy(x_hbm.at[pl.ds(base, _BPW)], x_v)

    for j in range(_BPW // _L):
        sl = pl.ds(j * _L, _L)
        xs = x_v[sl]
        xf = xs.astype(jnp.float32)
        for ci, c in enumerate(_CYCLES):
            q = (xf * (1.0 / c)).astype(jnp.int32)
            r = xs - q * c
            r = jnp.where(r < 0, r + c, r)
            r = jnp.where(r >= c, r - c, r)
            idx_refs[ci][sl] = r + (1 + _OFF[ci] // 128)

    plsc.subcore_barrier()
    pltpu.sync_copy(sh_tabs, ltab)

    writes = {}
    step = 0
    for ci in range(4):
        for h in range(_NCHUNK):
            p = step % 2
            if step >= 2:
                for w in writes[step - 2]:
                    w.wait()
            sbuf, cbuf = sbufs[p], cbufs[p]

            def grp(m, carry, ci=ci, h=h, sbuf=sbuf, cbuf=cbuf):
                g = m // 4
                kk = m % 4
                lane64 = lax.iota(jnp.int32, _L) * _C_DIM
                idxv = idx_refs[ci][pl.ds(h * _CH + g * _L, _L)]
                a = idxv * 128 + kk * _L
                a2 = a + _C_DIM
                w = lane64 + g * (_L * _C_DIM) + kk * _L
                for _k in range(_L):
                    plsc.store_scatter(sbuf, [w], plsc.load_gather(ltab, [a]))
                    plsc.store_scatter(cbuf, [w], plsc.load_gather(ltab, [a2]))
                    a = a + 1
                    a2 = a2 + 1
                    w = w + 1
                return carry

            lax.fori_loop(0, (_CH // _L) * 4, grp, 0)

            off = (base + h * _CH) * _C_DIM
            wa = pltpu.make_async_copy(
                sbuf, outs[2 * ci].at[pl.ds(off, _SLOT)], wsems[p])
            wb = pltpu.make_async_copy(
                cbuf, outs[2 * ci + 1].at[pl.ds(off, _SLOT)], wsems[p])
            wa.start()
            wb.start()
            writes[step] = (wa, wb)
            step += 1

    for s in (step - 2, step - 1):
        for w in writes[s]:
            w.wait()


@functools.partial(
    pl.kernel,
    out_type=[jax.ShapeDtypeStruct((_BATCH * _C_DIM,), jnp.float32)] * 8,
    mesh=plsc.VectorSubcoreMesh(core_axis_name="c", subcore_axis_name="s"),
    scratch_types=[
        pltpu.VMEM((_BPW,), jnp.int32),
        pltpu.VMEM((_BPW,), jnp.int32),
        pltpu.VMEM((_BPW,), jnp.int32),
        pltpu.VMEM((_BPW,), jnp.int32),
        pltpu.VMEM((_BPW,), jnp.int32),
        pltpu.VMEM((_TAB_LEN,), jnp.float32),
        pltpu.VMEM((_SLOT,), jnp.float32),
        pltpu.VMEM((_SLOT,), jnp.float32),
        pltpu.VMEM((_SLOT,), jnp.float32),
        pltpu.VMEM((_SLOT,), jnp.float32),
        pltpu.VMEM_SHARED((_TAB_LEN,), jnp.float32),
        pltpu.SemaphoreType.DMA,
        pltpu.SemaphoreType.DMA,
    ],
    compiler_params=pltpu.CompilerParams(needs_layout_passes=False),
)
def _cae_sc(*refs):
    _cae_body(*refs)


def kernel(x, W0, W1, W2, W3, W4, W5, W6, W7):
    x = x.astype(jnp.int32)
    tabs = jnp.concatenate([
        jnp.concatenate([Ws, Wc], axis=1).reshape(-1)
        for Ws, Wc in ((W0, W4), (W1, W5), (W2, W6), (W3, W7))
    ])
    o = _cae_sc(x, tabs)
    return tuple(r.reshape(_BATCH, _C_DIM) for r in o)


# speedup vs baseline: 2.0919x; 2.0919x over previous
import functools

import jax
import jax.numpy as jnp
from jax import lax
from jax.experimental import pallas as pl
from jax.experimental.pallas import tpu as pltpu
from jax.experimental.pallas import tpu_sc as plsc

_CYCLES = (7, 30, 91, 365)
_C_DIM = 64
_BATCH = 16384
_NC = 2
_NS = 16
_L = 16
_BPW = _BATCH // (_NC * _NS)
_CH = 128
_NCHUNK = _BPW // _CH
_ROWS = tuple(c + 1 for c in _CYCLES)
_OFF = tuple(128 * sum(_ROWS[:i]) for i in range(4))
_TAB_LEN = 128 * sum(_ROWS)
_SLOT = _CH * _C_DIM


def _cae_body(x_hbm, tabs_hbm,
              o0, o1, o2, o3, o4, o5, o6, o7,
              x_v, i0, i1, i2, i3, ltab, sb_a, sb_b, cb_a, cb_b,
              sh_tabs, ws0, ws1):
    outs = (o0, o1, o2, o3, o4, o5, o6, o7)
    idx_refs = (i0, i1, i2, i3)
    sbufs = (sb_a, sb_b)
    cbufs = (cb_a, cb_b)
    wsems = (ws0, ws1)

    c_id = lax.axis_index("c")
    s_id = lax.axis_index("s")
    base = (c_id * _NS + s_id) * _BPW

    @pl.when(s_id == 0)
    def _stage_tables():
        pltpu.sync_copy(tabs_hbm, sh_tabs)

    pltpu.sync_copy(x_hbm.at[pl.ds(base, _BPW)], x_v)

    for j in range(_BPW // _L):
        sl = pl.ds(j * _L, _L)
        xs = x_v[sl]
        xf = xs.astype(jnp.float32)
        for ci, c in enumerate(_CYCLES):
            q = (xf * (1.0 / c)).astype(jnp.int32)
            r = xs - q * c
            r = jnp.where(r < 0, r + c, r)
            r = jnp.where(r >= c, r - c, r)
            idx_refs[ci][sl] = r + (1 + _OFF[ci] // 128)

    plsc.subcore_barrier()
    pltpu.sync_copy(sh_tabs, ltab)

    writes = {}
    step = 0
    for ci in range(4):
        for h in range(_NCHUNK):
            p = step % 2
            if step >= 2:
                for w in writes[step - 2]:
                    w.wait()
            sbuf, cbuf = sbufs[p], cbufs[p]

            def grp(r, carry, ci=ci, h=h, sbuf=sbuf, cbuf=cbuf):
                sidx = idx_refs[ci][pl.ds(h * _CH + r, _L)][0]
                abase = sidx * 128
                wbase = r * _C_DIM
                for kk in range(_C_DIM // _L):
                    sbuf[pl.ds(wbase + kk * _L, _L)] = (
                        ltab[pl.ds(abase + kk * _L, _L)])
                    cbuf[pl.ds(wbase + kk * _L, _L)] = (
                        ltab[pl.ds(abase + _C_DIM + kk * _L, _L)])
                return carry

            lax.fori_loop(0, _CH, grp, 0)

            off = (base + h * _CH) * _C_DIM
            wa = pltpu.make_async_copy(
                sbuf, outs[2 * ci].at[pl.ds(off, _SLOT)], wsems[p])
            wb = pltpu.make_async_copy(
                cbuf, outs[2 * ci + 1].at[pl.ds(off, _SLOT)], wsems[p])
            wa.start()
            wb.start()
            writes[step] = (wa, wb)
            step += 1

    for s in (step - 2, step - 1):
        for w in writes[s]:
            w.wait()


@functools.partial(
    pl.kernel,
    out_type=[jax.ShapeDtypeStruct((_BATCH * _C_DIM,), jnp.float32)] * 8,
    mesh=plsc.VectorSubcoreMesh(core_axis_name="c", subcore_axis_name="s"),
    scratch_types=[
        pltpu.VMEM((_BPW,), jnp.int32),
        pltpu.VMEM((_BPW + _L,), jnp.int32),
        pltpu.VMEM((_BPW + _L,), jnp.int32),
        pltpu.VMEM((_BPW + _L,), jnp.int32),
        pltpu.VMEM((_BPW + _L,), jnp.int32),
        pltpu.VMEM((_TAB_LEN,), jnp.float32),
        pltpu.VMEM((_SLOT,), jnp.float32),
        pltpu.VMEM((_SLOT,), jnp.float32),
        pltpu.VMEM((_SLOT,), jnp.float32),
        pltpu.VMEM((_SLOT,), jnp.float32),
        pltpu.VMEM_SHARED((_TAB_LEN,), jnp.float32),
        pltpu.SemaphoreType.DMA,
        pltpu.SemaphoreType.DMA,
    ],
    compiler_params=pltpu.CompilerParams(needs_layout_passes=False),
)
def _cae_sc(*refs):
    _cae_body(*refs)


def kernel(x, W0, W1, W2, W3, W4, W5, W6, W7):
    x = x.astype(jnp.int32)
    tabs = jnp.concatenate([
        jnp.concatenate([Ws, Wc], axis=1).reshape(-1)
        for Ws, Wc in ((W0, W4), (W1, W5), (W2, W6), (W3, W7))
    ])
    o = _cae_sc(x, tabs)
    return tuple(r.reshape(_BATCH, _C_DIM) for r in o)


# speedup vs baseline: 2.2473x; 1.0743x over previous
import functools

import jax
import jax.numpy as jnp
from jax import lax
from jax.experimental import pallas as pl
from jax.experimental.pallas import tpu as pltpu
from jax.experimental.pallas import tpu_sc as plsc

_CYCLES = (7, 30, 91, 365)
_C_DIM = 64
_BATCH = 16384
_NC = 2
_NS = 16
_L = 16
_BPW = _BATCH // (_NC * _NS)
_CH = 128
_NCHUNK = _BPW // _CH
_ROWS = tuple(c + 1 for c in _CYCLES)
_OFF = tuple(128 * sum(_ROWS[:i]) for i in range(4))
_TAB_LEN = 128 * sum(_ROWS)
_SLOT = _CH * _C_DIM


def _cae_body(x_hbm, tabs_hbm,
              o0, o1, o2, o3, o4, o5, o6, o7,
              x_v, i0, i1, i2, i3, ltab, sb_a, sb_b, cb_a, cb_b,
              sh_tabs, ws0, ws1):
    outs = (o0, o1, o2, o3, o4, o5, o6, o7)
    idx_refs = (i0, i1, i2, i3)
    sbufs = (sb_a, sb_b)
    cbufs = (cb_a, cb_b)
    wsems = (ws0, ws1)

    c_id = lax.axis_index("c")
    s_id = lax.axis_index("s")
    base = (c_id * _NS + s_id) * _BPW

    @pl.when(s_id == 0)
    def _stage_tables():
        pltpu.sync_copy(tabs_hbm, sh_tabs)

    pltpu.sync_copy(x_hbm.at[pl.ds(base, _BPW)], x_v)

    for j in range(_BPW // _L):
        sl = pl.ds(j * _L, _L)
        xs = x_v[sl]
        xf = xs.astype(jnp.float32)
        for ci, c in enumerate(_CYCLES):
            q = (xf * (1.0 / c)).astype(jnp.int32)
            r = xs - q * c
            r = jnp.where(r < 0, r + c, r)
            r = jnp.where(r >= c, r - c, r)
            idx_refs[ci][sl] = r + (1 + _OFF[ci] // 128)

    plsc.subcore_barrier()
    pltpu.sync_copy(sh_tabs, ltab)

    writes = {}
    step = 0
    for ci in range(4):
        for h in range(_NCHUNK):
            p = step % 2
            if step >= 2:
                for w in writes[step - 2]:
                    w.wait()
            sbuf, cbuf = sbufs[p], cbufs[p]

            def grp(g, carry, ci=ci, h=h, sbuf=sbuf, cbuf=cbuf):
                idxv = idx_refs[ci][pl.ds(h * _CH + g * _L, _L)]
                wb = g * (_L * _C_DIM)

                def colgrp(kk, c2, idxv=idxv, wb=wb, sbuf=sbuf, cbuf=cbuf):
                    ko = kk * _L
                    for i in range(_L):
                        ab = idxv[i] * 128
                        sbuf[pl.ds(wb + i * _C_DIM + ko, _L)] = (
                            ltab[pl.ds(ab + ko, _L)])
                        cbuf[pl.ds(wb + i * _C_DIM + ko, _L)] = (
                            ltab[pl.ds(ab + _C_DIM + ko, _L)])
                    return c2

                lax.fori_loop(0, _C_DIM // _L, colgrp, 0)
                return carry

            lax.fori_loop(0, _CH // _L, grp, 0)

            off = (base + h * _CH) * _C_DIM
            wa = pltpu.make_async_copy(
                sbuf, outs[2 * ci].at[pl.ds(off, _SLOT)], wsems[p])
            wb = pltpu.make_async_copy(
                cbuf, outs[2 * ci + 1].at[pl.ds(off, _SLOT)], wsems[p])
            wa.start()
            wb.start()
            writes[step] = (wa, wb)
            step += 1

    for s in (step - 2, step - 1):
        for w in writes[s]:
            w.wait()


@functools.partial(
    pl.kernel,
    out_type=[jax.ShapeDtypeStruct((_BATCH * _C_DIM,), jnp.float32)] * 8,
    mesh=plsc.VectorSubcoreMesh(core_axis_name="c", subcore_axis_name="s"),
    scratch_types=[
        pltpu.VMEM((_BPW,), jnp.int32),
        pltpu.VMEM((_BPW + _L,), jnp.int32),
        pltpu.VMEM((_BPW + _L,), jnp.int32),
        pltpu.VMEM((_BPW + _L,), jnp.int32),
        pltpu.VMEM((_BPW + _L,), jnp.int32),
        pltpu.VMEM((_TAB_LEN,), jnp.float32),
        pltpu.VMEM((_SLOT,), jnp.float32),
        pltpu.VMEM((_SLOT,), jnp.float32),
        pltpu.VMEM((_SLOT,), jnp.float32),
        pltpu.VMEM((_SLOT,), jnp.float32),
        pltpu.VMEM_SHARED((_TAB_LEN,), jnp.float32),
        pltpu.SemaphoreType.DMA,
        pltpu.SemaphoreType.DMA,
    ],
    compiler_params=pltpu.CompilerParams(needs_layout_passes=False),
)
def _cae_sc(*refs):
    _cae_body(*refs)


def kernel(x, W0, W1, W2, W3, W4, W5, W6, W7):
    x = x.astype(jnp.int32)
    tabs = jnp.concatenate([
        jnp.concatenate([Ws, Wc], axis=1).reshape(-1)
        for Ws, Wc in ((W0, W4), (W1, W5), (W2, W6), (W3, W7))
    ])
    o = _cae_sc(x, tabs)
    return tuple(r.reshape(_BATCH, _C_DIM) for r in o)
